# baseline (speedup 1.0000x reference)
import jax
import jax.numpy as jnp
import numpy as np
from jax.experimental import pallas as pl
from jax.experimental.pallas import tpu as pltpu

_D = 4096
_E = 64
_K = 8
_S = 2
_BLOCK = 1024


def _gate_kernel(x_ref, w_ref, b_ref, probs_ref, idx_ref, logits_ref):
    logits = jnp.dot(x_ref[...], w_ref[...], preferred_element_type=jnp.float32)
    logits = logits + b_ref[...]
    logits_ref[...] = logits

    tr = logits.T
    bsz = tr.shape[1]
    rowf = jax.lax.broadcasted_iota(jnp.int32, tr.shape, 0).astype(jnp.float32)
    work = jnp.where(rowf >= _S, tr, -jnp.inf)
    vals, idxs = [], []
    for _ in range(_K):
        m = jnp.max(work, axis=0, keepdims=True)
        im = jnp.min(jnp.where(work == m, rowf, np.float32(_E)),
                     axis=0, keepdims=True)
        vals.append(m)
        idxs.append(im)
        work = jnp.where(rowf == im, -jnp.inf, work)

    spec_idx = jnp.concatenate(idxs, axis=0).astype(jnp.int32)
    tv = jnp.concatenate([tr[:_S, :]] + vals, axis=0)
    shared_idx = jax.lax.broadcasted_iota(jnp.int32, (_S, bsz), 0)
    ti = jnp.concatenate([shared_idx, spec_idx], axis=0)

    mx = jnp.max(tv, axis=0, keepdims=True)
    e = jnp.exp(tv - mx)
    p = e / jnp.sum(e, axis=0, keepdims=True)

    pad = jnp.zeros((16 - _S - _K, bsz), jnp.float32)
    probs_ref[...] = jnp.concatenate([p, pad], axis=0).T[:, : _S + _K]
    ipad = jnp.zeros((16 - _S - _K, bsz), jnp.int32)
    idx_ref[...] = jnp.concatenate([ti, ipad], axis=0).T[:, : _S + _K]


def kernel(inputs, W, b):
    n = inputs.shape[0]
    grid = (n // _BLOCK,)
    probs, idx, logits = pl.pallas_call(
        _gate_kernel,
        grid=grid,
        in_specs=[
            pl.BlockSpec((_BLOCK, _D), lambda i: (i, 0)),
            pl.BlockSpec((_D, _E), lambda i: (0, 0)),
            pl.BlockSpec((1, _E), lambda i: (0, 0)),
        ],
        out_specs=[
            pl.BlockSpec((_BLOCK, _S + _K), lambda i: (i, 0)),
            pl.BlockSpec((_BLOCK, _S + _K), lambda i: (i, 0)),
            pl.BlockSpec((_BLOCK, _E), lambda i: (i, 0)),
        ],
        out_shape=[
            jax.ShapeDtypeStruct((n, _S + _K), jnp.float32),
            jax.ShapeDtypeStruct((n, _S + _K), jnp.int32),
            jax.ShapeDtypeStruct((n, _E), jnp.float32),
        ],
        compiler_params=pltpu.CompilerParams(
            dimension_semantics=("parallel",),
        ),
    )(inputs, W, b.reshape(1, _E))
    return probs, idx, logits
